# Initial kernel scaffold; baseline (speedup 1.0000x reference)
import jax
import jax.numpy as jnp
from jax.experimental import pallas as pl


def kernel(protein_pos, protein_atom_feature, ligand_pos, ligand_atom_feature, batch_protein, batch_ligand, output_kind, enc_ligand_feature, enc_node_feature, enc_graph_feature, params):
    raise NotImplementedError("write your pallas kernel here")


# speedup vs baseline: 2.8995x; 2.8995x over previous
import functools

import jax
import jax.numpy as jnp
from jax import lax
from jax.experimental import pallas as pl
from jax.experimental.pallas import tpu as pltpu
from jax.experimental.pallas import tpu_sc as plsc

H = 128
NUM_RBF = 20
K = 16
CUTOFF = 10.0
OUT_DIM = 2
B = 32
N_RAW = 6600
NPAD = 6656
NBLK = NPAD // 128
NCH = 13
CW = 512
E_PAD = NPAD * K


def _embed_body(x_ref, w_ref, o_ref):
    o_ref[...] = jnp.dot(x_ref[...], w_ref[...],
                         preferred_element_type=jnp.float32)


def _embed(x, w):
    f = x.shape[1]
    return pl.pallas_call(
        _embed_body,
        grid=(NBLK,),
        in_specs=[pl.BlockSpec((128, f), lambda i: (i, 0)),
                  pl.BlockSpec((f, H), lambda i: (0, 0))],
        out_specs=pl.BlockSpec((128, H), lambda i: (i, 0)),
        out_shape=jax.ShapeDtypeStruct((NPAD, H), jnp.float32),
    )(x, w)


def _knn_body(pos_blk_ref, pos_ch_ref, bat_ch_ref, bat_row_ref,
              idx_ref, d_ref, s_ref):
    i = pl.program_id(0)
    pos_blk = pos_blk_ref[...]
    x2r = jnp.sum(pos_blk * pos_blk, axis=1, keepdims=True)
    brow = bat_row_ref[...]
    rowid = i * 128 + lax.broadcasted_iota(jnp.int32, (128, 1), 0)

    def fill(c, _):
        pc = pos_ch_ref[c]
        x2c = jnp.sum(pc * pc, axis=0, keepdims=True)
        dt = jnp.dot(pos_blk, pc, preferred_element_type=jnp.float32)
        d2 = x2r + x2c - 2.0 * dt
        bc = bat_ch_ref[c]
        col = c * CW + lax.broadcasted_iota(jnp.int32, (128, CW), 1)
        bad = (bc != brow) | (col == rowid)
        s_ref[c] = jnp.where(bad, jnp.inf, d2)
        return 0

    lax.fori_loop(0, NCH, fill, 0)

    big = jnp.float32(1e9)

    def extract(t, _):
        def scan_min(c, carry):
            m, a = carry
            ch = s_ref[c]
            mn = jnp.min(ch, axis=1, keepdims=True)
            colf = (c * CW
                    + lax.broadcasted_iota(jnp.int32, (128, CW), 1)
                    ).astype(jnp.float32)
            am = jnp.min(jnp.where(ch == mn, colf, big),
                         axis=1, keepdims=True)
            better = mn < m
            return (jnp.where(better, mn, m), jnp.where(better, am, a))

        m0 = jnp.full((128, 1), jnp.inf, jnp.float32)
        a0 = jnp.zeros((128, 1), jnp.float32)
        m, a = lax.fori_loop(0, NCH, scan_min, (m0, a0))

        def mask_out(c, _):
            ch = s_ref[c]
            colf = (c * CW
                    + lax.broadcasted_iota(jnp.int32, (128, CW), 1)
                    ).astype(jnp.float32)
            s_ref[c] = jnp.where(colf == a, jnp.inf, ch)
            return 0

        lax.fori_loop(0, NCH, mask_out, 0)
        idx_ref[t] = a.astype(jnp.int32)
        d_ref[t] = jnp.sqrt(jnp.maximum(m, 0.0) + 1e-12)
        return 0

    lax.fori_loop(0, K, extract, 0)


def _knn(pos_pad, pos_chunks, bat_chunks, bat_rows):
    return pl.pallas_call(
        _knn_body,
        grid=(NBLK,),
        in_specs=[pl.BlockSpec((128, 8), lambda i: (i, 0)),
                  pl.BlockSpec((NCH, 8, CW), lambda i: (0, 0, 0)),
                  pl.BlockSpec((NCH, 1, CW), lambda i: (0, 0, 0)),
                  pl.BlockSpec((128, 1), lambda i: (i, 0))],
        out_specs=[pl.BlockSpec((K, 128, 1), lambda i: (0, i, 0)),
                   pl.BlockSpec((K, 128, 1), lambda i: (0, i, 0))],
        out_shape=[jax.ShapeDtypeStruct((K, NPAD, 1), jnp.int32),
                   jax.ShapeDtypeStruct((K, NPAD, 1), jnp.float32)],
        scratch_shapes=[pltpu.VMEM((NCH, 128, CW), jnp.float32)],
    )(pos_pad, pos_chunks, bat_chunks, bat_rows)


_SC_CH = 416


def _make_sc_gather():
    info = plsc.get_sparse_core_info()
    nw = info.num_cores * info.num_subcores
    b_per_w = E_PAD // nw
    iters = b_per_w // _SC_CH
    mesh = plsc.VectorSubcoreMesh(core_axis_name="c", subcore_axis_name="s")

    @functools.partial(
        pl.kernel, mesh=mesh,
        out_type=jax.ShapeDtypeStruct((E_PAD, H), jnp.float32),
        scratch_types=[pltpu.VMEM((_SC_CH,), jnp.int32),
                       pltpu.VMEM((_SC_CH, H), jnp.float32),
                       pltpu.SemaphoreType.DMA],
    )
    def gather(h_hbm, idx_hbm, out_hbm, idx_v, rows_v, sem):
        wid = lax.axis_index("s") * info.num_cores + lax.axis_index("c")
        base = wid * b_per_w

        def body(i, _):
            off = base + i * _SC_CH
            pltpu.sync_copy(idx_hbm.at[pl.ds(off, _SC_CH)], idx_v)
            pltpu.async_copy(h_hbm.at[idx_v], rows_v, sem).wait()
            pltpu.sync_copy(rows_v, out_hbm.at[pl.ds(off, _SC_CH)])
            return 0

        lax.fori_loop(0, iters, body, 0)

    return gather


_sc_gather = None


def _gather_rows(h, idx):
    global _sc_gather
    if _sc_gather is None:
        _sc_gather = _make_sc_gather()
    return _sc_gather(h, idx)


def _layer_body(h_ref, hs_ref, d_ref,
                we1d_ref, we1s_ref, we1r_ref, be1_ref,
                we2_ref, be2_ref,
                wn1h_ref, wn1a_ref, bn1_ref, wn2_ref, bn2_ref,
                lng_ref, lnb_ref, o_ref):
    h_blk = h_ref[...]
    hs = hs_ref[...]
    d = d_ref[...]

    step = CUTOFF / (NUM_RBF - 1)
    coeff = -0.5 / (step * step)
    offi = lax.broadcasted_iota(jnp.int32, (1, 32), 1)
    offs = jnp.where(offi < NUM_RBF, offi.astype(jnp.float32) * step, 1e6)
    rbf = jnp.exp(coeff * (d - offs) ** 2)

    t = jnp.dot(h_blk, we1d_ref[...], preferred_element_type=jnp.float32)
    t_rep = jnp.broadcast_to(t[:, None, :], (128, K, H)).reshape(128 * K, H)
    m1 = (t_rep
          + jnp.dot(hs, we1s_ref[...], preferred_element_type=jnp.float32)
          + jnp.dot(rbf, we1r_ref[...], preferred_element_type=jnp.float32)
          + be1_ref[...])
    m1 = jnp.maximum(m1, 0.0)
    m2 = jnp.dot(m1, we2_ref[...], preferred_element_type=jnp.float32)
    m2 = jnp.maximum(m2 + be2_ref[...], 0.0)
    m2 = m2 * (d <= CUTOFF).astype(jnp.float32)
    agg = jnp.sum(m2.reshape(128, K, H), axis=1)

    u = (jnp.dot(h_blk, wn1h_ref[...], preferred_element_type=jnp.float32)
         + jnp.dot(agg, wn1a_ref[...], preferred_element_type=jnp.float32)
         + bn1_ref[...])
    u = jnp.maximum(u, 0.0)
    u = jnp.dot(u, wn2_ref[...], preferred_element_type=jnp.float32) \
        + bn2_ref[...]
    r = h_blk + u
    mu = jnp.mean(r, axis=-1, keepdims=True)
    var = jnp.mean((r - mu) ** 2, axis=-1, keepdims=True)
    o_ref[...] = (r - mu) / jnp.sqrt(var + 1e-5) * lng_ref[...] \
        + lnb_ref[...]


def _layer(h, hs, d_flat, lw):
    row = lambda a: a.reshape(1, -1)
    args = (h, hs, d_flat,
            lw['We1'][:H], lw['We1'][H:2 * H],
            jnp.pad(lw['We1'][2 * H:], ((0, 32 - NUM_RBF), (0, 0))),
            row(lw['be1']),
            lw['We2'], row(lw['be2']),
            lw['Wn1'][:H], lw['Wn1'][H:], row(lw['bn1']),
            lw['Wn2'], row(lw['bn2']),
            row(lw['ln_g']), row(lw['ln_b']))
    const = lambda shp: pl.BlockSpec(shp, lambda i: tuple(0 for _ in shp))
    return pl.pallas_call(
        _layer_body,
        grid=(NBLK,),
        in_specs=[pl.BlockSpec((128, H), lambda i: (i, 0)),
                  pl.BlockSpec((128 * K, H), lambda i: (i, 0)),
                  pl.BlockSpec((128 * K, 1), lambda i: (i, 0)),
                  const((H, H)), const((H, H)), const((32, H)),
                  const((1, H)), const((H, H)), const((1, H)),
                  const((H, H)), const((H, H)), const((1, H)),
                  const((H, H)), const((1, H)),
                  const((1, H)), const((1, H))],
        out_specs=pl.BlockSpec((128, H), lambda i: (i, 0)),
        out_shape=jax.ShapeDtypeStruct((NPAD, H), jnp.float32),
    )(*args)


def _final_body(h_ref, enc_ref, bat_ref,
                w1h_ref, w1e_ref, b1_ref, w2_ref, b2_ref,
                encg_ref, wo1h_ref, wo1g_ref, bo1_ref,
                wo2_ref, bo2_ref, kind_ref, o_ref, acc_ref):
    i = pl.program_id(0)

    he = (jnp.dot(h_ref[...], w1h_ref[...],
                  preferred_element_type=jnp.float32)
          + jnp.dot(enc_ref[...], w1e_ref[...],
                    preferred_element_type=jnp.float32)
          + b1_ref[...])
    he = jnp.maximum(he, 0.0)
    h2 = jnp.dot(he, w2_ref[...], preferred_element_type=jnp.float32) \
        + b2_ref[...]

    oh = (bat_ref[...] ==
          lax.broadcasted_iota(jnp.int32, (1, B), 1)).astype(jnp.float32)
    contrib = lax.dot_general(oh, h2, (((0,), (0,)), ((), ())),
                              preferred_element_type=jnp.float32)

    @pl.when(i == 0)
    def _():
        acc_ref[...] = jnp.zeros_like(acc_ref)

    acc_ref[...] += contrib

    @pl.when(i == NBLK - 1)
    def _():
        pre = acc_ref[...]
        z = (jnp.dot(pre, wo1h_ref[...], preferred_element_type=jnp.float32)
             + jnp.dot(encg_ref[...], wo1g_ref[...],
                       preferred_element_type=jnp.float32)
             + bo1_ref[...])
        z = jax.nn.softplus(z) - jnp.log(2.0)
        o2 = jnp.dot(z, wo2_ref[...], preferred_element_type=jnp.float32) \
            + bo2_ref[...]
        sel = (kind_ref[...] - 1 ==
               lax.broadcasted_iota(jnp.int32, (B, OUT_DIM), 1)
               ).astype(jnp.float32)
        o_ref[...] = jnp.sum(o2 * sel, axis=1, keepdims=True)


def _final(h, enc_pad, bat_rows, p, encg_pad, kind):
    row = lambda a: a.reshape(1, -1)
    args = (h, enc_pad, bat_rows,
            p['Wenc1'][:H], p['Wenc1'][H:], row(p['benc1']),
            p['Wenc2'], row(p['benc2']),
            encg_pad, p['Wo1'][:H],
            jnp.pad(p['Wo1'][H:], ((0, 4), (0, 0))), row(p['bo1']),
            p['Wo2'], row(p['bo2']), kind.reshape(B, 1).astype(jnp.int32))
    const = lambda shp: pl.BlockSpec(shp, lambda i: tuple(0 for _ in shp))
    return pl.pallas_call(
        _final_body,
        grid=(NBLK,),
        in_specs=[pl.BlockSpec((128, H), lambda i: (i, 0)),
                  pl.BlockSpec((128, 16), lambda i: (i, 0)),
                  pl.BlockSpec((128, 1), lambda i: (i, 0)),
                  const((H, H)), const((16, H)), const((1, H)),
                  const((H, H)), const((1, H)),
                  const((B, 8)), const((H, H)), const((8, H)),
                  const((1, H)), const((H, OUT_DIM)), const((1, OUT_DIM)),
                  const((B, 1))],
        out_specs=pl.BlockSpec((B, 1), lambda i: (0, 0)),
        out_shape=jax.ShapeDtypeStruct((B, 1), jnp.float32),
        scratch_shapes=[pltpu.VMEM((B, H), jnp.float32)],
    )(*args)


def kernel(protein_pos, protein_atom_feature, ligand_pos, ligand_atom_feature,
           batch_protein, batch_ligand, output_kind,
           enc_ligand_feature, enc_node_feature, enc_graph_feature, params):
    np_, nl = protein_pos.shape[0], ligand_pos.shape[0]
    pf = protein_atom_feature.shape[1]
    lf = ligand_atom_feature.shape[1] + enc_ligand_feature.shape[1]

    batch_all = jnp.concatenate([batch_protein, batch_ligand])
    order = jnp.argsort(batch_all, stable=True)
    batch_ctx = batch_all[order]
    pos = jnp.concatenate([protein_pos, ligand_pos], axis=0)[order]

    xp = jnp.concatenate([
        protein_atom_feature, jnp.zeros((np_, lf), jnp.float32),
        jnp.ones((np_, 1), jnp.float32), jnp.zeros((np_, 1), jnp.float32)],
        axis=1)
    xl = jnp.concatenate([
        jnp.zeros((nl, pf), jnp.float32), ligand_atom_feature,
        enc_ligand_feature,
        jnp.zeros((nl, 1), jnp.float32), jnp.ones((nl, 1), jnp.float32)],
        axis=1)
    x = jnp.concatenate([xp, xl], axis=0)[order]
    fdim = pf + lf + 2
    fpad = ((fdim + 7) // 8) * 8
    x = jnp.pad(x, ((0, NPAD - N_RAW), (0, fpad - fdim)))
    wcat = jnp.concatenate([
        params['Wp'], params['Wl'],
        params['bp'].reshape(1, H), params['bl'].reshape(1, H),
        jnp.zeros((fpad - fdim, H), jnp.float32)], axis=0)

    pos_pad = jnp.pad(pos, ((0, NPAD - N_RAW), (0, 8 - 3)))
    bat_pad = jnp.concatenate([
        batch_ctx.astype(jnp.int32),
        64 + jnp.arange(NPAD - N_RAW, dtype=jnp.int32)])
    pos_chunks = pos_pad.reshape(NCH, CW, 8).transpose(0, 2, 1)
    bat_chunks = bat_pad.reshape(NCH, 1, CW)
    bat_rows = bat_pad.reshape(NPAD, 1)

    h = _embed(x, wcat)
    idx16, d16 = _knn(pos_pad, pos_chunks, bat_chunks, bat_rows)
    idx_flat = idx16.reshape(K, NPAD).T.reshape(-1)
    d_flat = d16.reshape(K, NPAD).T.reshape(E_PAD, 1)

    for lw in params['enc_layers']:
        hs = _gather_rows(h, idx_flat)
        h = _layer(h, hs, d_flat, lw)

    enc_pad = jnp.pad(enc_node_feature, ((0, NPAD - N_RAW), (0, 0)))
    encg_pad = jnp.pad(enc_graph_feature, ((0, 0), (0, 4)))
    return _final(h, enc_pad, bat_rows, params, encg_pad, output_kind)


# speedup vs baseline: 8.3263x; 2.8716x over previous
import functools

import jax
import jax.numpy as jnp
from jax import lax
from jax.experimental import pallas as pl
from jax.experimental.pallas import tpu as pltpu
from jax.experimental.pallas import tpu_sc as plsc

H = 128
NUM_RBF = 20
K = 16
CUTOFF = 10.0
OUT_DIM = 2
B = 32
N_RAW = 6600
NPAD = 6656
NBLK = NPAD // 128
NCH = 13
CW = 512
E_PAD = NPAD * K


def _embed_body(x_ref, w_ref, o_ref):
    o_ref[...] = jnp.dot(x_ref[...], w_ref[...],
                         preferred_element_type=jnp.float32)


def _embed(x, w):
    f = x.shape[1]
    return pl.pallas_call(
        _embed_body,
        grid=(NBLK,),
        in_specs=[pl.BlockSpec((128, f), lambda i: (i, 0)),
                  pl.BlockSpec((f, H), lambda i: (0, 0))],
        out_specs=pl.BlockSpec((128, H), lambda i: (i, 0)),
        out_shape=jax.ShapeDtypeStruct((NPAD, H), jnp.float32),
    )(x, w)


_COLBITS = 0x1FFF


def _knn_body(clo_ref, chi_ref, pos_blk_ref, pos_ch_ref, bat_ch_ref,
              bat_row_ref, idx_ref, d_ref, s_ref, d2_ref):
    i = pl.program_id(0)
    clo = clo_ref[i]
    chi = chi_ref[i]
    pos_blk = pos_blk_ref[...]
    x2r = jnp.sum(pos_blk * pos_blk, axis=1, keepdims=True)
    brow = bat_row_ref[...]
    rowid = i * 128 + lax.broadcasted_iota(jnp.int32, (128, 1), 0)

    def fill(c, _):
        pc = pos_ch_ref[c]
        x2c = jnp.sum(pc * pc, axis=0, keepdims=True)
        dt = jnp.dot(pos_blk, pc, preferred_element_type=jnp.float32)
        d2 = jnp.maximum(x2r + x2c - 2.0 * dt, 0.0)
        bc = bat_ch_ref[c]
        col = c * CW + lax.broadcasted_iota(jnp.int32, (128, CW), 1)
        bad = (bc != brow) | (col == rowid)
        d2 = jnp.where(bad, jnp.inf, d2)
        key = (lax.bitcast_convert_type(d2, jnp.int32)
               & ~_COLBITS) | col
        s_ref[c] = key
        d2_ref[c] = d2
        return 0

    lax.fori_loop(clo, chi + 1, fill, 0)

    imax = jnp.int32(0x7FFFFFFF)

    def extract(t, kt):
        def scan_min(c, m):
            ch = s_ref[c]
            cand = jnp.where(ch > kt, ch, imax)
            return jnp.minimum(m, jnp.min(cand, axis=1, keepdims=True))

        m0 = jnp.full((128, 1), imax, jnp.int32)
        m = lax.fori_loop(clo, chi + 1, scan_min, m0)

        def scan_d2(c, a):
            hit = jnp.where(s_ref[c] == m, d2_ref[c], jnp.inf)
            return jnp.minimum(a, jnp.min(hit, axis=1, keepdims=True))

        a0 = jnp.full((128, 1), jnp.inf, jnp.float32)
        d2x = lax.fori_loop(clo, chi + 1, scan_d2, a0)
        idx_ref[t] = jnp.minimum(m & _COLBITS, NPAD - 1)
        d_ref[t] = jnp.sqrt(jnp.maximum(d2x, 0.0) + 1e-12)
        return m

    kt0 = jnp.full((128, 1), jnp.int32(-0x80000000), jnp.int32)
    lax.fori_loop(0, K, extract, kt0)


def _knn(pos_pad, pos_chunks, bat_chunks, bat_rows, clo, chi):
    return pl.pallas_call(
        _knn_body,
        grid=(NBLK,),
        in_specs=[pl.BlockSpec(memory_space=pltpu.SMEM),
                  pl.BlockSpec(memory_space=pltpu.SMEM),
                  pl.BlockSpec((128, 8), lambda i: (i, 0)),
                  pl.BlockSpec((NCH, 8, CW), lambda i: (0, 0, 0)),
                  pl.BlockSpec((NCH, 1, CW), lambda i: (0, 0, 0)),
                  pl.BlockSpec((128, 1), lambda i: (i, 0))],
        out_specs=[pl.BlockSpec((K, 128, 1), lambda i: (0, i, 0)),
                   pl.BlockSpec((K, 128, 1), lambda i: (0, i, 0))],
        out_shape=[jax.ShapeDtypeStruct((K, NPAD, 1), jnp.int32),
                   jax.ShapeDtypeStruct((K, NPAD, 1), jnp.float32)],
        scratch_shapes=[pltpu.VMEM((NCH, 128, CW), jnp.int32),
                        pltpu.VMEM((NCH, 128, CW), jnp.float32)],
    )(clo, chi, pos_pad, pos_chunks, bat_chunks, bat_rows)


_SC_CH = 416
GD = H


def _make_sc_gather():
    info = plsc.get_sparse_core_info()
    nw = info.num_cores * info.num_subcores
    b_per_w = E_PAD // nw
    iters = b_per_w // _SC_CH
    mesh = plsc.VectorSubcoreMesh(core_axis_name="c", subcore_axis_name="s")

    @functools.partial(
        pl.kernel, mesh=mesh,
        out_type=jax.ShapeDtypeStruct((E_PAD, GD), jnp.float32),
        scratch_types=[pltpu.VMEM((_SC_CH,), jnp.int32),
                       pltpu.VMEM((_SC_CH, GD), jnp.float32),
                       pltpu.SemaphoreType.DMA],
    )
    def gather(h_hbm, idx_hbm, out_hbm, idx_v, rows_v, sem):
        wid = lax.axis_index("s") * info.num_cores + lax.axis_index("c")
        base = wid * b_per_w

        def body(i, _):
            off = base + i * _SC_CH
            pltpu.sync_copy(idx_hbm.at[pl.ds(off, _SC_CH)], idx_v)
            pltpu.async_copy(h_hbm.at[idx_v], rows_v, sem).wait()
            pltpu.sync_copy(rows_v, out_hbm.at[pl.ds(off, _SC_CH)])
            return 0

        lax.fori_loop(0, iters, body, 0)

    return gather


_sc_gather = None


def _gather_rows(h, idx):
    global _sc_gather
    if _sc_gather is None:
        _sc_gather = _make_sc_gather()
    return _sc_gather(h, idx)


def _layer_body(h_ref, hs_ref, d_ref,
                we1d_ref, we1s_ref, we1r_ref, be1_ref,
                we2_ref, be2_ref,
                wn1h_ref, wn1a_ref, bn1_ref, wn2_ref, bn2_ref,
                lng_ref, lnb_ref, o_ref):
    h_blk = h_ref[...]
    hs = hs_ref[...]
    d = d_ref[...]

    step = CUTOFF / (NUM_RBF - 1)
    coeff = -0.5 / (step * step)
    offi = lax.broadcasted_iota(jnp.int32, (1, 32), 1)
    offs = jnp.where(offi < NUM_RBF, offi.astype(jnp.float32) * step, 1e6)
    rbf = jnp.exp(coeff * (d - offs) ** 2)

    t = jnp.dot(h_blk, we1d_ref[...], preferred_element_type=jnp.float32)
    t_rep = jnp.broadcast_to(t[:, None, :], (128, K, H)).reshape(128 * K, H)
    m1 = (t_rep
          + jnp.dot(hs, we1s_ref[...], preferred_element_type=jnp.float32)
          + jnp.dot(rbf, we1r_ref[...], preferred_element_type=jnp.float32)
          + be1_ref[...])
    m1 = jnp.maximum(m1, 0.0)
    m2 = jnp.dot(m1, we2_ref[...], preferred_element_type=jnp.float32)
    m2 = jnp.maximum(m2 + be2_ref[...], 0.0)
    m2 = m2 * (d <= CUTOFF).astype(jnp.float32)
    agg = jnp.sum(m2.reshape(128, K, H), axis=1)

    u = (jnp.dot(h_blk, wn1h_ref[...], preferred_element_type=jnp.float32)
         + jnp.dot(agg, wn1a_ref[...], preferred_element_type=jnp.float32)
         + bn1_ref[...])
    u = jnp.maximum(u, 0.0)
    u = jnp.dot(u, wn2_ref[...], preferred_element_type=jnp.float32) \
        + bn2_ref[...]
    r = h_blk + u
    mu = jnp.mean(r, axis=-1, keepdims=True)
    var = jnp.mean((r - mu) ** 2, axis=-1, keepdims=True)
    o_ref[...] = (r - mu) / jnp.sqrt(var + 1e-5) * lng_ref[...] \
        + lnb_ref[...]


def _layer(h, hs, d_flat, lw):
    row = lambda a: a.reshape(1, -1)
    args = (h, hs, d_flat,
            lw['We1'][:H], lw['We1'][H:2 * H],
            jnp.pad(lw['We1'][2 * H:], ((0, 32 - NUM_RBF), (0, 0))),
            row(lw['be1']),
            lw['We2'], row(lw['be2']),
            lw['Wn1'][:H], lw['Wn1'][H:], row(lw['bn1']),
            lw['Wn2'], row(lw['bn2']),
            row(lw['ln_g']), row(lw['ln_b']))
    const = lambda shp: pl.BlockSpec(shp, lambda i: tuple(0 for _ in shp))
    return pl.pallas_call(
        _layer_body,
        grid=(NBLK,),
        in_specs=[pl.BlockSpec((128, H), lambda i: (i, 0)),
                  pl.BlockSpec((128 * K, H), lambda i: (i, 0)),
                  pl.BlockSpec((128 * K, 1), lambda i: (i, 0)),
                  const((H, H)), const((H, H)), const((32, H)),
                  const((1, H)), const((H, H)), const((1, H)),
                  const((H, H)), const((H, H)), const((1, H)),
                  const((H, H)), const((1, H)),
                  const((1, H)), const((1, H))],
        out_specs=pl.BlockSpec((128, H), lambda i: (i, 0)),
        out_shape=jax.ShapeDtypeStruct((NPAD, H), jnp.float32),
    )(*args)


def _final_body(h_ref, enc_ref, bat_ref,
                w1h_ref, w1e_ref, b1_ref, w2_ref, b2_ref,
                encg_ref, wo1h_ref, wo1g_ref, bo1_ref,
                wo2_ref, bo2_ref, kind_ref, o_ref, acc_ref):
    i = pl.program_id(0)

    he = (jnp.dot(h_ref[...], w1h_ref[...],
                  preferred_element_type=jnp.float32)
          + jnp.dot(enc_ref[...], w1e_ref[...],
                    preferred_element_type=jnp.float32)
          + b1_ref[...])
    he = jnp.maximum(he, 0.0)
    h2 = jnp.dot(he, w2_ref[...], preferred_element_type=jnp.float32) \
        + b2_ref[...]

    oh = (bat_ref[...] ==
          lax.broadcasted_iota(jnp.int32, (1, B), 1)).astype(jnp.float32)
    contrib = lax.dot_general(oh, h2, (((0,), (0,)), ((), ())),
                              preferred_element_type=jnp.float32)

    @pl.when(i == 0)
    def _():
        acc_ref[...] = jnp.zeros_like(acc_ref)

    acc_ref[...] += contrib

    @pl.when(i == NBLK - 1)
    def _():
        pre = acc_ref[...]
        z = (jnp.dot(pre, wo1h_ref[...], preferred_element_type=jnp.float32)
             + jnp.dot(encg_ref[...], wo1g_ref[...],
                       preferred_element_type=jnp.float32)
             + bo1_ref[...])
        z = jax.nn.softplus(z) - jnp.log(2.0)
        o2 = jnp.dot(z, wo2_ref[...], preferred_element_type=jnp.float32) \
            + bo2_ref[...]
        sel = (kind_ref[...] - 1 ==
               lax.broadcasted_iota(jnp.int32, (B, OUT_DIM), 1)
               ).astype(jnp.float32)
        o_ref[...] = jnp.sum(o2 * sel, axis=1, keepdims=True)


def _final(h, enc_pad, bat_rows, p, encg_pad, kind):
    row = lambda a: a.reshape(1, -1)
    args = (h, enc_pad, bat_rows,
            p['Wenc1'][:H], p['Wenc1'][H:], row(p['benc1']),
            p['Wenc2'], row(p['benc2']),
            encg_pad, p['Wo1'][:H],
            jnp.pad(p['Wo1'][H:], ((0, 4), (0, 0))), row(p['bo1']),
            p['Wo2'], row(p['bo2']), kind.reshape(B, 1).astype(jnp.int32))
    const = lambda shp: pl.BlockSpec(shp, lambda i: tuple(0 for _ in shp))
    return pl.pallas_call(
        _final_body,
        grid=(NBLK,),
        in_specs=[pl.BlockSpec((128, H), lambda i: (i, 0)),
                  pl.BlockSpec((128, 16), lambda i: (i, 0)),
                  pl.BlockSpec((128, 1), lambda i: (i, 0)),
                  const((H, H)), const((16, H)), const((1, H)),
                  const((H, H)), const((1, H)),
                  const((B, 8)), const((H, H)), const((8, H)),
                  const((1, H)), const((H, OUT_DIM)), const((1, OUT_DIM)),
                  const((B, 1))],
        out_specs=pl.BlockSpec((B, 1), lambda i: (0, 0)),
        out_shape=jax.ShapeDtypeStruct((B, 1), jnp.float32),
        scratch_shapes=[pltpu.VMEM((B, H), jnp.float32)],
    )(*args)


def kernel(protein_pos, protein_atom_feature, ligand_pos, ligand_atom_feature,
           batch_protein, batch_ligand, output_kind,
           enc_ligand_feature, enc_node_feature, enc_graph_feature, params):
    np_, nl = protein_pos.shape[0], ligand_pos.shape[0]
    pf = protein_atom_feature.shape[1]
    lf = ligand_atom_feature.shape[1] + enc_ligand_feature.shape[1]

    batch_all = jnp.concatenate([batch_protein, batch_ligand])
    order = jnp.argsort(batch_all, stable=True)
    batch_ctx = batch_all[order]
    pos = jnp.concatenate([protein_pos, ligand_pos], axis=0)[order]

    xp = jnp.concatenate([
        protein_atom_feature, jnp.zeros((np_, lf), jnp.float32),
        jnp.ones((np_, 1), jnp.float32), jnp.zeros((np_, 1), jnp.float32)],
        axis=1)
    xl = jnp.concatenate([
        jnp.zeros((nl, pf), jnp.float32), ligand_atom_feature,
        enc_ligand_feature,
        jnp.zeros((nl, 1), jnp.float32), jnp.ones((nl, 1), jnp.float32)],
        axis=1)
    x = jnp.concatenate([xp, xl], axis=0)[order]
    fdim = pf + lf + 2
    fpad = ((fdim + 7) // 8) * 8
    x = jnp.pad(x, ((0, NPAD - N_RAW), (0, fpad - fdim)))
    wcat = jnp.concatenate([
        params['Wp'], params['Wl'],
        params['bp'].reshape(1, H), params['bl'].reshape(1, H),
        jnp.zeros((fpad - fdim, H), jnp.float32)], axis=0)

    pos_pad = jnp.pad(pos, ((0, NPAD - N_RAW), (0, 8 - 3)))
    bat_pad = jnp.concatenate([
        batch_ctx.astype(jnp.int32),
        64 + jnp.arange(NPAD - N_RAW, dtype=jnp.int32)])
    pos_chunks = pos_pad.reshape(NCH, CW, 8).transpose(0, 2, 1)
    bat_chunks = bat_pad.reshape(NCH, 1, CW)
    bat_rows = bat_pad.reshape(NPAD, 1)

    blk = jnp.arange(NBLK)
    bmin = bat_pad[blk * 128]
    bmax = bat_pad[blk * 128 + 127]
    lo = jnp.searchsorted(bat_pad, bmin, side='left')
    hi = jnp.searchsorted(bat_pad, bmax, side='right')
    clo = (lo // CW).astype(jnp.int32)
    chi = ((hi - 1) // CW).astype(jnp.int32)

    h = _embed(x, wcat)
    idx16, d16 = _knn(pos_pad, pos_chunks, bat_chunks, bat_rows, clo, chi)
    idx_flat = idx16.reshape(K, NPAD).T.reshape(-1)
    d_flat = d16.reshape(K, NPAD).T.reshape(E_PAD, 1)

    for lw in params['enc_layers']:
        hs = _gather_rows(h, idx_flat)
        h = _layer(h, hs, d_flat, lw)

    enc_pad = jnp.pad(enc_node_feature, ((0, NPAD - N_RAW), (0, 0)))
    encg_pad = jnp.pad(enc_graph_feature, ((0, 0), (0, 4)))
    return _final(h, enc_pad, bat_rows, params, encg_pad, output_kind)
